# baseline (speedup 1.0000x reference)
import functools

import jax
import jax.numpy as jnp
from jax import lax
from jax.experimental import pallas as pl
from jax.experimental.pallas import tpu as pltpu
from jax.experimental.pallas import tpu_sc as plsc

N_NODES = 10000
N_EDGES = 160000
NC = 2
NS = 16
L = 16

EPT = N_EDGES // NS
CE = 2000
NCHUNK = EPT // CE
NVC = CE // L
KSUB = 128


def _make_agg(W, CH, NPASS, with_bias=False):
  S = W // 128
  KEDGE = KSUB // S
  STRIPE = CH // NS
  NBLK = STRIPE // 16
  ZR = 4 * S
  NZ = STRIPE * S // ZR
  CPAD = CE + KEDGE + L
  assert STRIPE % 16 == 0 and (STRIPE * S) % ZR == 0
  mesh = plsc.VectorSubcoreMesh(core_axis_name="c", subcore_axis_name="s")

  scratch = [
      pltpu.VMEM((CE,), jnp.int32),
      pltpu.VMEM((CE,), jnp.int32),
      pltpu.VMEM((CPAD,), jnp.int32),
      pltpu.VMEM((CPAD,), jnp.int32),
      pltpu.VMEM((2, KEDGE), jnp.int32),
      pltpu.VMEM((2, KSUB), jnp.int32),
      pltpu.VMEM((2, KEDGE, S, 128), jnp.float32),
      pltpu.VMEM((ZR, 128), jnp.float32),
      pltpu.VMEM_SHARED(((CH + 16) * S, 128), jnp.float32),
      pltpu.SemaphoreType.DMA,
      pltpu.SemaphoreType.DMA,
  ]
  if with_bias:
    scratch.append(pltpu.VMEM((W,), jnp.float32))

  def body(h_hbm, src_hbm, dst_hbm, *rest):
    if with_bias:
      b_hbm = rest[0]
      rest = rest[1:]
    out_hbm = rest[0]
    (src_c, dst_c, csrc, cdst, idx_s, idx_d, rows_v, zbuf, accum,
     semg, sems) = rest[1:12]
    if with_bias:
      bvec = rest[12]

    cid = lax.axis_index("c")
    sid = lax.axis_index("s")
    lane = lax.iota(jnp.int32, L)

    if with_bias:
      pltpu.sync_copy(b_hbm, bvec)
    zero = jnp.zeros((L,), jnp.float32)

    def init_row(r, _):
      for k in range(128 // L):
        if with_bias:
          zbuf[r, pl.ds(k * L, L)] = bvec[pl.ds(k * L, L)]
        else:
          zbuf[r, pl.ds(k * L, L)] = zero
      return 0

    lax.fori_loop(0, ZR, init_row, 0)

    def expand(q, parity, base_sub):
      for j in range(KEDGE // L):
        e = q * KEDGE + j * L + lane
        idx_s[parity, pl.ds(j * L, L)] = plsc.load_gather(csrc, [e])
      for j in range(KSUB // L):
        lanes = j * L + lane
        e = base_sub + q * KEDGE + lanes // S
        sub = lanes - (lanes // S) * S
        dv = plsc.load_gather(cdst, [e])
        idx_d[parity, pl.ds(j * L, L)] = dv * S + sub

    def start_gather(parity):
      return pltpu.async_copy(
          h_hbm.at[idx_s.at[parity]], rows_v.at[parity], semg)

    def start_scatter(parity):
      return pltpu.async_copy(
          rows_v.at[parity].reshape(KSUB, 128),
          accum.at[idx_d.at[parity]], sems, add=True)

    def wait_scatter(parity):
      pltpu.make_async_copy(
          rows_v.at[parity].reshape(KSUB, 128),
          accum.at[idx_d.at[parity]], sems).wait()

    for p in range(NPASS):
      chunk_base = p * NC * CH + cid * CH

      for b in range(NZ):
        pltpu.sync_copy(
            zbuf, accum.at[pl.ds(sid * STRIPE * S + b * ZR, ZR)])
      plsc.subcore_barrier()

      for c in range(NCHUNK):
        e0 = sid * EPT + c * CE
        pltpu.sync_copy(src_hbm.at[pl.ds(e0, CE)], src_c)
        pltpu.sync_copy(dst_hbm.at[pl.ds(e0, CE)], dst_c)

        def scan_body(i, n):
          sv = src_c[pl.ds(i * L, L)]
          dv = dst_c[pl.ds(i * L, L)]
          m = (dv >= chunk_base) & (dv < chunk_base + CH)
          mi = m.astype(jnp.int32)
          pos = n + plsc.cumsum(mi) - 1
          plsc.store_scatter(csrc, [pos], sv, mask=m)
          plsc.store_scatter(cdst, [pos], dv - chunk_base, mask=m)
          return n + jnp.sum(mi)

        n = lax.fori_loop(0, NVC, scan_body, jnp.int32(0))
        nops = (n + (KEDGE - 1)) // KEDGE
        for t in range(KEDGE // L + 1):
          csrc[pl.ds(n + t * L, L)] = jnp.zeros((L,), jnp.int32)
          cdst[pl.ds(n + t * L, L)] = jnp.full((L,), CH, jnp.int32)

        @pl.when(nops > 0)
        def _():
          expand(0, 0, 0)
          start_gather(0)

          def op_body(q, _):
            parity = q & 1

            @pl.when(q + 1 < nops)
            def _():
              @pl.when(q >= 1)
              def _():
                wait_scatter(1 - parity)
              expand(q + 1, 1 - parity, 0)
              start_gather(1 - parity)

            pltpu.make_async_copy(
                h_hbm.at[idx_s.at[parity]], rows_v.at[parity], semg).wait()
            start_scatter(parity)
            return 0

          lax.fori_loop(0, nops, op_body, 0)
          wait_scatter((nops - 1) & 1)

          @pl.when(nops >= 2)
          def _():
            wait_scatter(nops & 1)

      plsc.subcore_barrier()

      for b in range(NBLK):
        r0 = sid * STRIPE + b * 16
        node0 = chunk_base + r0
        @pl.when(node0 < N_NODES)
        def _():
          pltpu.sync_copy(accum.at[pl.ds(r0 * S, 16 * S)],
                          out_hbm.at[pl.ds(node0 * S, 16 * S)])
      if p != NPASS - 1:
        plsc.subcore_barrier()

  return pl.kernel(
      body,
      out_type=jax.ShapeDtypeStruct((N_NODES * S, 128), jnp.float32),
      mesh=mesh,
      scratch_types=scratch,
      compiler_params=pltpu.CompilerParams(needs_layout_passes=False),
  )


_agg_256 = _make_agg(256, CH=2560, NPASS=2)
_agg_1024 = _make_agg(1024, CH=1024, NPASS=5)
_agg_128b = _make_agg(128, CH=5120, NPASS=1, with_bias=True)


def _mm_body(x_ref, w_ref, b_ref, o_ref, *, relu):
  acc = jnp.dot(x_ref[...], w_ref[...], preferred_element_type=jnp.float32)
  acc = acc + b_ref[...]
  if relu:
    acc = jnp.maximum(acc, 0.0)
  o_ref[...] = acc


def _mm_nobias_body(x_ref, w_ref, o_ref):
  o_ref[...] = jnp.dot(x_ref[...], w_ref[...], preferred_element_type=jnp.float32)


def _matmul(x, w, b=None, relu=False, block_m=2000):
  M, K = x.shape
  K2, N = w.shape
  grid = (M // block_m,)
  if b is not None:
    return pl.pallas_call(
        functools.partial(_mm_body, relu=relu),
        grid=grid,
        in_specs=[
            pl.BlockSpec((block_m, K), lambda i: (i, 0)),
            pl.BlockSpec((K, N), lambda i: (0, 0)),
            pl.BlockSpec((N,), lambda i: (0,)),
        ],
        out_specs=pl.BlockSpec((block_m, N), lambda i: (i, 0)),
        out_shape=jax.ShapeDtypeStruct((M, N), jnp.float32),
    )(x, w, b)
  return pl.pallas_call(
      _mm_nobias_body,
      grid=grid,
      in_specs=[
          pl.BlockSpec((block_m, K), lambda i: (i, 0)),
          pl.BlockSpec((K, N), lambda i: (0, 0)),
      ],
      out_specs=pl.BlockSpec((block_m, N), lambda i: (i, 0)),
      out_shape=jax.ShapeDtypeStruct((M, N), jnp.float32),
  )(x, w)


def kernel(features, edge_index, W0, b0, W1, b1, W2, b2):
  src = edge_index[0]
  dst = edge_index[1]
  a0 = _agg_256(features.reshape(-1, 2, 128), src, dst).reshape(N_NODES, 256)
  h1 = _matmul(a0, W0, b0, relu=True)
  a1 = _agg_1024(h1.reshape(-1, 8, 128), src, dst).reshape(N_NODES, 1024)
  h2 = _matmul(a1, W1, b1, relu=True)
  g = _matmul(h2, W2)
  out = _agg_128b(g.reshape(-1, 1, 128), src, dst, b2)
  return out


# speedup vs baseline: 1.1370x; 1.1370x over previous
import functools

import jax
import jax.numpy as jnp
from jax import lax
from jax.experimental import pallas as pl
from jax.experimental.pallas import tpu as pltpu
from jax.experimental.pallas import tpu_sc as plsc

N_NODES = 10000
N_EDGES = 160000
NC = 2
NS = 16
L = 16

EPT = N_EDGES // NS
CE = 2000
NCHUNK = EPT // CE
NVC = CE // L
KSUB = 128


def _make_agg(W, CH, NPASS, with_bias=False):
  S = W // 128
  KEDGE = KSUB // S
  STRIPE = CH // NS
  NBLK = STRIPE // 16
  ZR = 32
  NZ = STRIPE * S // ZR
  CPAD = CE + KEDGE + L
  PK = 16384
  assert STRIPE % 16 == 0 and (STRIPE * S) % ZR == 0
  mesh = plsc.VectorSubcoreMesh(core_axis_name="c", subcore_axis_name="s")

  scratch = [
      pltpu.VMEM((CE,), jnp.int32),
      pltpu.VMEM((CE,), jnp.int32),
      pltpu.VMEM((CPAD,), jnp.int32),
      pltpu.VMEM((2, KEDGE), jnp.int32),
      pltpu.VMEM((2, KSUB), jnp.int32),
      pltpu.VMEM((2, KEDGE, S, 128), jnp.float32),
      pltpu.VMEM((ZR, 128), jnp.float32),
      pltpu.VMEM_SHARED(((CH + 16) * S, 128), jnp.float32),
      pltpu.SemaphoreType.DMA,
      pltpu.SemaphoreType.DMA,
  ]
  if with_bias:
    scratch.append(pltpu.VMEM((W,), jnp.float32))

  def body(h_hbm, src_hbm, dst_hbm, *rest):
    if with_bias:
      b_hbm = rest[0]
      rest = rest[1:]
    out_hbm = rest[0]
    (src_c, dst_c, cpk, idx_s, idx_d, rows_v, zbuf, accum,
     semg, sems) = rest[1:11]
    if with_bias:
      bvec = rest[11]

    cid = lax.axis_index("c")
    sid = lax.axis_index("s")
    lane = lax.iota(jnp.int32, L)

    if with_bias:
      pltpu.sync_copy(b_hbm, bvec)
    zero = jnp.zeros((L,), jnp.float32)

    def init_row(r, _):
      for k in range(128 // L):
        if with_bias:
          zbuf[r, pl.ds(k * L, L)] = bvec[pl.ds(k * L, L)]
        else:
          zbuf[r, pl.ds(k * L, L)] = zero
      return 0

    lax.fori_loop(0, ZR, init_row, 0)

    def expand(q, parity, base_sub):
      for j in range(KEDGE // L):
        e = q * KEDGE + j * L + lane
        idx_s[parity, pl.ds(j * L, L)] = plsc.load_gather(cpk, [e]) % PK
      for j in range(KSUB // L):
        lanes = j * L + lane
        e = base_sub + q * KEDGE + lanes // S
        sub = lanes - (lanes // S) * S
        dv = plsc.load_gather(cpk, [e]) // PK
        idx_d[parity, pl.ds(j * L, L)] = dv * S + sub

    def start_gather(parity):
      return pltpu.async_copy(
          h_hbm.at[idx_s.at[parity]], rows_v.at[parity], semg)

    def start_scatter(parity):
      return pltpu.async_copy(
          rows_v.at[parity].reshape(KSUB, 128),
          accum.at[idx_d.at[parity]], sems, add=True)

    def wait_scatter(parity):
      pltpu.make_async_copy(
          rows_v.at[parity].reshape(KSUB, 128),
          accum.at[idx_d.at[parity]], sems).wait()

    for p in range(NPASS):
      chunk_base = p * NC * CH + cid * CH

      for b in range(NZ):
        pltpu.sync_copy(
            zbuf, accum.at[pl.ds(sid * STRIPE * S + b * ZR, ZR)])
      plsc.subcore_barrier()

      for c in range(NCHUNK):
        e0 = sid * EPT + c * CE
        pltpu.sync_copy(src_hbm.at[pl.ds(e0, CE)], src_c)
        pltpu.sync_copy(dst_hbm.at[pl.ds(e0, CE)], dst_c)

        def scan_body(i, n):
          sv = src_c[pl.ds(i * L, L)]
          dv = dst_c[pl.ds(i * L, L)]
          m = (dv >= chunk_base) & (dv < chunk_base + CH)
          mi = m.astype(jnp.int32)
          pos = n + plsc.cumsum(mi) - 1
          plsc.store_scatter(cpk, [pos], (dv - chunk_base) * PK + sv, mask=m)
          return pos[L - 1] + 1

        n = lax.fori_loop(0, NVC, scan_body, jnp.int32(0))
        nops = (n + (KEDGE - 1)) // KEDGE
        for t in range(KEDGE // L + 1):
          cpk[pl.ds(n + t * L, L)] = jnp.full((L,), CH * PK, jnp.int32)

        @pl.when(nops > 0)
        def _():
          expand(0, 0, 0)
          start_gather(0)

          def op_body(q, _):
            parity = q & 1

            @pl.when(q + 1 < nops)
            def _():
              @pl.when(q >= 1)
              def _():
                wait_scatter(1 - parity)
              expand(q + 1, 1 - parity, 0)
              start_gather(1 - parity)

            pltpu.make_async_copy(
                h_hbm.at[idx_s.at[parity]], rows_v.at[parity], semg).wait()
            start_scatter(parity)
            return 0

          lax.fori_loop(0, nops, op_body, 0)
          wait_scatter((nops - 1) & 1)

          @pl.when(nops >= 2)
          def _():
            wait_scatter(nops & 1)

      plsc.subcore_barrier()

      for b in range(NBLK):
        r0 = sid * STRIPE + b * 16
        node0 = chunk_base + r0
        @pl.when(node0 < N_NODES)
        def _():
          pltpu.sync_copy(accum.at[pl.ds(r0 * S, 16 * S)],
                          out_hbm.at[pl.ds(node0 * S, 16 * S)])
      if p != NPASS - 1:
        plsc.subcore_barrier()

  return pl.kernel(
      body,
      out_type=jax.ShapeDtypeStruct((N_NODES * S, 128), jnp.float32),
      mesh=mesh,
      scratch_types=scratch,
      compiler_params=pltpu.CompilerParams(needs_layout_passes=False),
  )


_agg_256 = _make_agg(256, CH=5120, NPASS=1)
_agg_1024 = _make_agg(1024, CH=1024, NPASS=5)
_agg_128b = _make_agg(128, CH=5120, NPASS=1, with_bias=True)


def _mm_body(x_ref, w_ref, b_ref, o_ref, *, relu):
  acc = jnp.dot(x_ref[...], w_ref[...], preferred_element_type=jnp.float32)
  acc = acc + b_ref[...]
  if relu:
    acc = jnp.maximum(acc, 0.0)
  o_ref[...] = acc


def _mm_nobias_body(x_ref, w_ref, o_ref):
  o_ref[...] = jnp.dot(x_ref[...], w_ref[...], preferred_element_type=jnp.float32)


def _matmul(x, w, b=None, relu=False, block_m=2000):
  M, K = x.shape
  K2, N = w.shape
  grid = (M // block_m,)
  if b is not None:
    return pl.pallas_call(
        functools.partial(_mm_body, relu=relu),
        grid=grid,
        in_specs=[
            pl.BlockSpec((block_m, K), lambda i: (i, 0)),
            pl.BlockSpec((K, N), lambda i: (0, 0)),
            pl.BlockSpec((N,), lambda i: (0,)),
        ],
        out_specs=pl.BlockSpec((block_m, N), lambda i: (i, 0)),
        out_shape=jax.ShapeDtypeStruct((M, N), jnp.float32),
    )(x, w, b)
  return pl.pallas_call(
      _mm_nobias_body,
      grid=grid,
      in_specs=[
          pl.BlockSpec((block_m, K), lambda i: (i, 0)),
          pl.BlockSpec((K, N), lambda i: (0, 0)),
      ],
      out_specs=pl.BlockSpec((block_m, N), lambda i: (i, 0)),
      out_shape=jax.ShapeDtypeStruct((M, N), jnp.float32),
  )(x, w)


def kernel(features, edge_index, W0, b0, W1, b1, W2, b2):
  src = edge_index[0]
  dst = edge_index[1]
  a0 = _agg_256(features.reshape(-1, 2, 128), src, dst).reshape(N_NODES, 256)
  h1 = _matmul(a0, W0, b0, relu=True)
  a1 = _agg_1024(h1.reshape(-1, 8, 128), src, dst).reshape(N_NODES, 1024)
  h2 = _matmul(a1, W1, b1, relu=True)
  g = _matmul(h2, W2)
  out = _agg_128b(g.reshape(-1, 1, 128), src, dst, b2)
  return out


# speedup vs baseline: 1.1820x; 1.0396x over previous
import functools

import jax
import jax.numpy as jnp
from jax import lax
from jax.experimental import pallas as pl
from jax.experimental.pallas import tpu as pltpu
from jax.experimental.pallas import tpu_sc as plsc

N_NODES = 10000
N_EDGES = 160000
NC = 2
NS = 16
L = 16

EPT = N_EDGES // NS
CE = 2000
NCHUNK = EPT // CE
NVC = CE // L
KSUB = 128


def _make_agg(W, CH, NPASS, with_bias=False):
  S = W // 128
  KEDGE = KSUB // S
  STRIPE = CH // NS
  NBLK = STRIPE // 16
  ZR = 32
  NZ = STRIPE * S // ZR
  CPAD = CE + KEDGE + L
  PK = 16384
  assert STRIPE % 16 == 0 and (STRIPE * S) % ZR == 0
  mesh = plsc.VectorSubcoreMesh(core_axis_name="c", subcore_axis_name="s")

  scratch = [
      pltpu.VMEM((CE,), jnp.int32),
      pltpu.VMEM((CE,), jnp.int32),
      pltpu.VMEM((CPAD,), jnp.int32),
      pltpu.VMEM((2, KEDGE), jnp.int32),
      pltpu.VMEM((2, KSUB), jnp.int32),
      pltpu.VMEM((2, KEDGE, S, 128), jnp.float32),
      pltpu.VMEM((ZR, 128), jnp.float32),
      pltpu.VMEM_SHARED(((CH + 16) * S, 128), jnp.float32),
      pltpu.SemaphoreType.DMA,
      pltpu.SemaphoreType.DMA,
  ]
  if with_bias:
    scratch.append(pltpu.VMEM((W,), jnp.float32))

  def body(h_hbm, src_hbm, dst_hbm, *rest):
    if with_bias:
      b_hbm = rest[0]
      rest = rest[1:]
    out_hbm = rest[0]
    (src_c, dst_c, cpk, idx_s, idx_d, rows_v, zbuf, accum,
     semg, sems) = rest[1:11]
    if with_bias:
      bvec = rest[11]

    cid = lax.axis_index("c")
    sid = lax.axis_index("s")
    lane = lax.iota(jnp.int32, L)

    if with_bias:
      pltpu.sync_copy(b_hbm, bvec)
    zero = jnp.zeros((L,), jnp.float32)

    def init_row(r, _):
      for k in range(128 // L):
        if with_bias:
          zbuf[r, pl.ds(k * L, L)] = bvec[pl.ds(k * L, L)]
        else:
          zbuf[r, pl.ds(k * L, L)] = zero
      return 0

    lax.fori_loop(0, ZR, init_row, 0)

    def expand(q, parity, base_sub):
      for j in range(KEDGE // L):
        e = q * KEDGE + j * L + lane
        idx_s[parity, pl.ds(j * L, L)] = plsc.load_gather(cpk, [e]) % PK
      for j in range(KSUB // L):
        lanes = j * L + lane
        e = base_sub + q * KEDGE + lanes // S
        sub = lanes - (lanes // S) * S
        dv = plsc.load_gather(cpk, [e]) // PK
        idx_d[parity, pl.ds(j * L, L)] = dv * S + sub

    def start_gather(parity):
      return pltpu.async_copy(
          h_hbm.at[idx_s.at[parity]], rows_v.at[parity], semg)

    def start_scatter(parity):
      return pltpu.async_copy(
          rows_v.at[parity].reshape(KSUB, 128),
          accum.at[idx_d.at[parity]], sems, add=True)

    def wait_scatter(parity):
      pltpu.make_async_copy(
          rows_v.at[parity].reshape(KSUB, 128),
          accum.at[idx_d.at[parity]], sems).wait()

    for p in range(NPASS):
      chunk_base = p * NC * CH + cid * CH

      for b in range(NZ):
        pltpu.sync_copy(
            zbuf, accum.at[pl.ds(sid * STRIPE * S + b * ZR, ZR)])
      plsc.subcore_barrier()

      for c in range(NCHUNK):
        e0 = sid * EPT + c * CE
        pltpu.sync_copy(src_hbm.at[pl.ds(e0, CE)], src_c)
        pltpu.sync_copy(dst_hbm.at[pl.ds(e0, CE)], dst_c)

        def scan_body(i, n):
          sv = src_c[pl.ds(i * L, L)]
          dv = dst_c[pl.ds(i * L, L)]
          m = (dv >= chunk_base) & (dv < chunk_base + CH)
          mi = m.astype(jnp.int32)
          pos = n + plsc.cumsum(mi) - 1
          plsc.store_scatter(cpk, [pos], (dv - chunk_base) * PK + sv, mask=m)
          return pos[L - 1] + 1

        n = lax.fori_loop(0, NVC, scan_body, jnp.int32(0))
        nops = (n + (KEDGE - 1)) // KEDGE
        for t in range(KEDGE // L + 1):
          cpk[pl.ds(n + t * L, L)] = jnp.full((L,), CH * PK, jnp.int32)

        @pl.when(nops > 0)
        def _():
          expand(0, 0, 0)
          start_gather(0)

          def op_body(q, _):
            parity = q & 1

            @pl.when(q + 1 < nops)
            def _():
              @pl.when(q >= 1)
              def _():
                wait_scatter(1 - parity)
              expand(q + 1, 1 - parity, 0)
              start_gather(1 - parity)

            pltpu.make_async_copy(
                h_hbm.at[idx_s.at[parity]], rows_v.at[parity], semg).wait()
            start_scatter(parity)
            return 0

          lax.fori_loop(0, nops, op_body, 0)
          wait_scatter((nops - 1) & 1)

          @pl.when(nops >= 2)
          def _():
            wait_scatter(nops & 1)

      plsc.subcore_barrier()

      for b in range(NBLK):
        r0 = sid * STRIPE + b * 16
        node0 = chunk_base + r0
        @pl.when(node0 < N_NODES)
        def _():
          pltpu.sync_copy(accum.at[pl.ds(r0 * S, 16 * S)],
                          out_hbm.at[pl.ds(node0 * S, 16 * S)])
      if p != NPASS - 1:
        plsc.subcore_barrier()

  return pl.kernel(
      body,
      out_type=jax.ShapeDtypeStruct((N_NODES * S, 128), jnp.float32),
      mesh=mesh,
      scratch_types=scratch,
      compiler_params=pltpu.CompilerParams(needs_layout_passes=False),
  )


_agg_256 = _make_agg(256, CH=5120, NPASS=1)
_agg_1024 = _make_agg(1024, CH=1280, NPASS=4)
_agg_128b = _make_agg(128, CH=5120, NPASS=1, with_bias=True)


def _mm_body(x_ref, w_ref, b_ref, o_ref, *, relu):
  acc = jnp.dot(x_ref[...], w_ref[...], preferred_element_type=jnp.float32)
  acc = acc + b_ref[...]
  if relu:
    acc = jnp.maximum(acc, 0.0)
  o_ref[...] = acc


def _mm_nobias_body(x_ref, w_ref, o_ref):
  o_ref[...] = jnp.dot(x_ref[...], w_ref[...], preferred_element_type=jnp.float32)


def _matmul(x, w, b=None, relu=False, block_m=2000):
  M, K = x.shape
  K2, N = w.shape
  grid = (M // block_m,)
  if b is not None:
    return pl.pallas_call(
        functools.partial(_mm_body, relu=relu),
        grid=grid,
        in_specs=[
            pl.BlockSpec((block_m, K), lambda i: (i, 0)),
            pl.BlockSpec((K, N), lambda i: (0, 0)),
            pl.BlockSpec((N,), lambda i: (0,)),
        ],
        out_specs=pl.BlockSpec((block_m, N), lambda i: (i, 0)),
        out_shape=jax.ShapeDtypeStruct((M, N), jnp.float32),
    )(x, w, b)
  return pl.pallas_call(
      _mm_nobias_body,
      grid=grid,
      in_specs=[
          pl.BlockSpec((block_m, K), lambda i: (i, 0)),
          pl.BlockSpec((K, N), lambda i: (0, 0)),
      ],
      out_specs=pl.BlockSpec((block_m, N), lambda i: (i, 0)),
      out_shape=jax.ShapeDtypeStruct((M, N), jnp.float32),
  )(x, w)


def kernel(features, edge_index, W0, b0, W1, b1, W2, b2):
  src = edge_index[0]
  dst = edge_index[1]
  a0 = _agg_256(features.reshape(-1, 2, 128), src, dst).reshape(N_NODES, 256)
  h1 = _matmul(a0, W0, b0, relu=True)
  a1 = _agg_1024(h1.reshape(-1, 8, 128), src, dst).reshape(N_NODES, 1024)
  h2 = _matmul(a1, W1, b1, relu=True)
  g = _matmul(h2, W2)
  out = _agg_128b(g.reshape(-1, 1, 128), src, dst, b2)
  return out
